# baseline (speedup 1.0000x reference)
import functools

import jax
import jax.numpy as jnp
from jax import lax
from jax.experimental import pallas as pl
from jax.experimental.pallas import tpu as pltpu
from jax.experimental.pallas import tpu_sc as plsc


@functools.lru_cache(maxsize=None)
def _make_lookup(V, D, B):
    info = plsc.get_sparse_core_info()
    NC, NS = info.num_cores, info.num_subcores
    NW = NC * NS
    LW = 768
    N_FULL = V // LW
    TAIL = V - N_FULL * LW
    NWIN = N_FULL + (1 if TAIL else 0)
    N_K = (NWIN + NW - 1) // NW
    IC = 1024
    N_IC = B // IC
    STG = 2048
    PAD = STG
    mesh = plsc.VectorSubcoreMesh(core_axis_name="c", subcore_axis_name="s")

    @functools.partial(
        pl.kernel,
        mesh=mesh,
        out_type=jax.ShapeDtypeStruct((B * D + PAD,), jnp.float32),
        scratch_types=[
            pltpu.VMEM((D, LW), jnp.float32),
            pltpu.VMEM((D, LW), jnp.float32),
            pltpu.VMEM((D, 64), jnp.float32),
            pltpu.VMEM((B + 16,), jnp.int32),
            pltpu.VMEM((B + 16,), jnp.int32),
            pltpu.VMEM((B + 16,), jnp.int32),
            pltpu.VMEM((IC,), jnp.int32),
            pltpu.VMEM((IC,), jnp.int32),
            pltpu.VMEM((STG,), jnp.float32),
            pltpu.VMEM((STG,), jnp.float32),
            pltpu.VMEM((STG,), jnp.int32),
            pltpu.VMEM((STG,), jnp.int32),
            pltpu.SemaphoreType.DMA,
            pltpu.SemaphoreType.DMA,
            pltpu.SemaphoreType.DMA,
            pltpu.SemaphoreType.DMA,
            pltpu.SemaphoreType.DMA,
            pltpu.SemaphoreType.DMA,
        ],
        compiler_params=pltpu.CompilerParams(
            use_tc_tiling_on_sc=True, needs_layout_passes=False
        ),
    )
    def lookup(tt_hbm, idx_hbm, out_hbm, buf0, buf1, buf_tail,
               r_list, b_list, tmp_p, ic0, ic1, vs0, vs1, is0, is1,
               sw0, sw1, si0, si1, ss0, ss1):
        wid = lax.axis_index("s") * NC + lax.axis_index("c")
        c_iota = lax.iota(jnp.int32, 16)
        bufs = (buf0, buf1)
        sws = (sw0, sw1)
        ics = (ic0, ic1)
        sis = (si0, si1)
        vss = (vs0, vs1)
        iss = (is0, is1)
        sss = (ss0, ss1)

        pltpu.async_copy(idx_hbm.at[pl.ds(0, IC)], ic0, si0)
        cnt = jnp.int32(0)
        for ic in range(N_IC):
            par = ic % 2
            if ic + 1 < N_IC:
                pltpu.async_copy(idx_hbm.at[pl.ds((ic + 1) * IC, IC)],
                                 ics[1 - par], sis[1 - par])
            pltpu.make_async_copy(idx_hbm.at[pl.ds(ic * IC, IC)],
                                  ics[par], sis[par]).wait()
            cur = ics[par]
            b_base = jnp.int32(ic * IC)

            def scan_chunk(g, cnt, cur=cur, b_base=b_base):
                rv = cur[pl.ds(g * 16, 16)]
                w = rv // LW
                mine = (w & (NW - 1)) == wid
                bv = b_base + g * 16 + c_iota
                pos = cnt + plsc.cumsum(mine.astype(jnp.int32)) - 1
                plsc.store_scatter(r_list, [pos], rv, mask=mine)
                plsc.store_scatter(b_list, [pos], bv, mask=mine)
                return cnt + plsc.all_reduce_population_count(mine)[0]

            cnt = lax.fori_loop(0, IC // 16, scan_chunk, cnt)

        n_groups = (cnt + 15) >> 4

        def issue_window(w, par):
            @pl.when(w < N_FULL)
            def _():
                pltpu.async_copy(tt_hbm.at[:, pl.ds(w * LW, LW)],
                                 bufs[par], sws[par])
            @pl.when(w == N_FULL)
            def _():
                pltpu.async_copy(tt_hbm.at[:, pl.ds(N_FULL * LW, TAIL)],
                                 buf_tail, sws[par])

        def wait_window(w, par):
            @pl.when(w < N_FULL)
            def _():
                pltpu.make_async_copy(tt_hbm.at[:, pl.ds(w * LW, LW)],
                                      bufs[par], sws[par]).wait()
            @pl.when(w == N_FULL)
            def _():
                pltpu.make_async_copy(tt_hbm.at[:, pl.ds(N_FULL * LW, TAIL)],
                                      buf_tail, sws[par]).wait()
                for c in range(D):
                    for h in range(TAIL // 16):
                        bufs[par][c, pl.ds(h * 16, 16)] = (
                            buf_tail[c, pl.ds(h * 16, 16)])

        def scatter_desc(par):
            return pltpu.make_async_copy(
                vss[par], out_hbm.at[iss[par]], sss[par])

        def process_window(w, par):
            lo = w * LW
            hi = jnp.minimum(lo + LW, V)

            def match(g, nw):
                rv = r_list[pl.ds(g * 16, 16)]
                bv = b_list[pl.ds(g * 16, 16)]
                inb = (rv >= lo) & (rv < hi) & ((g * 16 + c_iota) < cnt)
                p = (bv << 10) | (rv - lo)
                pos = nw + plsc.cumsum(inb.astype(jnp.int32)) - 1
                plsc.store_scatter(tmp_p, [pos], p, mask=inb)
                return nw + plsc.all_reduce_population_count(inb)[0]

            nw = lax.fori_loop(0, n_groups, match, jnp.int32(0))
            nch = jnp.maximum((nw + 63) >> 6, 1)

            buf, vs, ist = bufs[par], vss[par], iss[par]

            def chunk(ch, _):
                base = ch * 64
                for grp in range(4):
                    pv = tmp_p[pl.ds(base + grp * 16, 16)]
                    for j in range(16):
                        e = grp * 16 + j
                        pos_e = base + e
                        p_j = pv[j]

                        @pl.when(pos_e < nw)
                        def _(p_j=p_j, e=e):
                            b_j = p_j >> 10
                            rloc = p_j & 1023
                            rsplat = lax.broadcast(rloc, (16,))
                            for h in range(2):
                                vals = plsc.load_gather(
                                    buf, [c_iota + h * 16, rsplat])
                                vs[pl.ds(e * 32 + h * 16, 16)] = vals
                                ist[pl.ds(e * 32 + h * 16, 16)] = (
                                    lax.broadcast(b_j * 32 + h * 16, (16,))
                                    + c_iota)

                        @pl.when(pos_e >= nw)
                        def _(e=e):
                            for h in range(2):
                                ist[pl.ds(e * 32 + h * 16, 16)] = (
                                    lax.broadcast(
                                        jnp.int32(B * D + e * 32 + h * 16),
                                        (16,)) + c_iota)
                @pl.when(ch < nch - 1)
                def _():
                    scatter_desc(par).start()
                    scatter_desc(par).wait()
                return 0

            lax.fori_loop(0, nch, chunk, 0)
            scatter_desc(par).start()

        issue_window(wid, 0)

        def body(k, _):
            par = k % 2
            w = wid + k * NW
            active = w <= N_FULL

            @pl.when(wid + (k + 1) * NW <= N_FULL)
            def _():
                for p in range(2):
                    @pl.when((k + 1) % 2 == p)
                    def _(p=p):
                        issue_window(wid + (k + 1) * NW, p)

            @pl.when((k >= 2) & (wid + (k - 2) * NW <= N_FULL))
            def _():
                for p in range(2):
                    @pl.when(k % 2 == p)
                    def _(p=p):
                        scatter_desc(p).wait()

            @pl.when(active)
            def _():
                for p in range(2):
                    @pl.when(par == p)
                    def _(p=p):
                        wait_window(w, p)
                        process_window(w, p)

            return 0

        lax.fori_loop(0, N_K + 2, body, 0)

    return lookup


@jax.jit
def kernel(species, embed_table):
    V, D = embed_table.shape
    (B,) = species.shape
    flat = _make_lookup(V, D, B)(embed_table.T, species.astype(jnp.int32))
    return flat[: B * D].reshape(B, D)


# speedup vs baseline: 152.2304x; 152.2304x over previous
import functools

import jax
import jax.numpy as jnp
from jax import lax
from jax.experimental import pallas as pl
from jax.experimental.pallas import tpu as pltpu
from jax.experimental.pallas import tpu_sc as plsc

_LW = 512


@functools.lru_cache(maxsize=None)
def _make_lookup(V, D, B):
    info = plsc.get_sparse_core_info()
    NC, NS = info.num_cores, info.num_subcores
    NW = NC * NS
    LW = _LW
    N_FULL = V // LW
    TAIL = V - N_FULL * LW
    NWIN = N_FULL + (1 if TAIL else 0)
    N_K = (NWIN + NW - 1) // NW
    IC = 1024
    N_IC = B // IC
    STG = 1024
    PAD = STG
    N_OUT = B * D + PAD
    mesh = plsc.VectorSubcoreMesh(core_axis_name="c", subcore_axis_name="s")

    @functools.partial(
        pl.kernel,
        mesh=mesh,
        out_type=(
            jax.ShapeDtypeStruct((N_OUT,), jnp.float32),
            jax.ShapeDtypeStruct((N_OUT,), jnp.float32),
        ),
        scratch_types=[
            pltpu.VMEM((D, LW), jnp.float32),
            pltpu.VMEM((D, LW), jnp.float32),
            pltpu.VMEM((D, 64), jnp.float32),
            pltpu.VMEM((B + 16,), jnp.int32),
            pltpu.VMEM((B + 16,), jnp.int32),
            pltpu.VMEM((IC,), jnp.int32),
            pltpu.VMEM((IC,), jnp.int32),
            pltpu.VMEM((STG,), jnp.float32),
            pltpu.VMEM((STG,), jnp.float32),
            pltpu.VMEM((STG,), jnp.int32),
            pltpu.VMEM((STG,), jnp.int32),
            pltpu.VMEM_SHARED((N_OUT,), jnp.float32),
            pltpu.SemaphoreType.DMA,
            pltpu.SemaphoreType.DMA,
            pltpu.SemaphoreType.DMA,
            pltpu.SemaphoreType.DMA,
            pltpu.SemaphoreType.DMA,
            pltpu.SemaphoreType.DMA,
        ],
        compiler_params=pltpu.CompilerParams(
            use_tc_tiling_on_sc=True, needs_layout_passes=False
        ),
    )
    def lookup(tt_hbm, idx_hbm, out0_hbm, out1_hbm, buf0, buf1, buf_tail,
               enc_list, tmp_p, ic0, ic1, vs0, vs1, is0, is1, sp_out,
               sw0, sw1, si0, si1, ss0, ss1):
        cid = lax.axis_index("c")
        wid = lax.axis_index("s") * NC + cid
        c_iota = lax.iota(jnp.int32, 16)
        bufs = (buf0, buf1)
        sws = (sw0, sw1)
        ics = (ic0, ic1)
        sis = (si0, si1)
        vss = (vs0, vs1)
        iss = (is0, is1)
        sss = (ss0, ss1)

        pltpu.async_copy(idx_hbm.at[pl.ds(0, IC)], ic0, si0)
        cnt = jnp.int32(0)
        for ic in range(N_IC):
            par = ic % 2
            if ic + 1 < N_IC:
                pltpu.async_copy(idx_hbm.at[pl.ds((ic + 1) * IC, IC)],
                                 ics[1 - par], sis[1 - par])
            pltpu.make_async_copy(idx_hbm.at[pl.ds(ic * IC, IC)],
                                  ics[par], sis[par]).wait()
            cur = ics[par]
            b_base = jnp.int32(ic * IC)

            def scan_chunk(g, cnt, cur=cur, b_base=b_base):
                rv = cur[pl.ds(g * 16, 16)]
                w = rv >> 9
                mine = (w & (NW - 1)) == wid
                bv = b_base + g * 16 + c_iota
                enc = ((w >> 5) << 24) | ((rv & 511) << 14) | bv
                pos = cnt + plsc.cumsum(mine.astype(jnp.int32)) - 1
                plsc.store_scatter(enc_list, [pos], enc, mask=mine)
                return cnt + plsc.all_reduce_population_count(mine)[0]

            cnt = lax.fori_loop(0, IC // 16, scan_chunk, cnt)

        n_groups = (cnt + 15) >> 4

        def issue_window(w, par):
            @pl.when(w < N_FULL)
            def _():
                pltpu.async_copy(tt_hbm.at[:, pl.ds(w * LW, LW)],
                                 bufs[par], sws[par])
            @pl.when(w == N_FULL)
            def _():
                pltpu.async_copy(tt_hbm.at[:, pl.ds(N_FULL * LW, TAIL)],
                                 buf_tail, sws[par])

        def wait_window(w, par):
            @pl.when(w < N_FULL)
            def _():
                pltpu.make_async_copy(tt_hbm.at[:, pl.ds(w * LW, LW)],
                                      bufs[par], sws[par]).wait()
            @pl.when(w == N_FULL)
            def _():
                pltpu.make_async_copy(tt_hbm.at[:, pl.ds(N_FULL * LW, TAIL)],
                                      buf_tail, sws[par]).wait()
                for c in range(D):
                    for h in range(TAIL // 16):
                        bufs[par][c, pl.ds(h * 16, 16)] = (
                            buf_tail[c, pl.ds(h * 16, 16)])

        def scatter_desc(par):
            return pltpu.make_async_copy(
                vss[par], sp_out.at[iss[par]], sss[par])

        def process_window(k, par):
            def match(g, nw):
                ev = enc_list[pl.ds(g * 16, 16)]
                inb = ((ev >> 24) == k) & ((g * 16 + c_iota) < cnt)
                pos = nw + plsc.cumsum(inb.astype(jnp.int32)) - 1
                plsc.store_scatter(tmp_p, [pos], ev, mask=inb)
                return nw + plsc.all_reduce_population_count(inb)[0]

            nw = lax.fori_loop(0, n_groups, match, jnp.int32(0))
            nch = jnp.maximum((nw + (STG // 32 - 1)) >> 5, 1)

            buf, vs, ist = bufs[par], vss[par], iss[par]

            def chunk(ch, _):
                base = ch * (STG // 32)
                for grp in range(STG // 32 // 16):
                    pv = tmp_p[pl.ds(base + grp * 16, 16)]
                    for j in range(16):
                        e = grp * 16 + j
                        pos_e = base + e
                        p_j = pv[j]

                        @pl.when(pos_e < nw)
                        def _(p_j=p_j, e=e):
                            b_j = p_j & 16383
                            rloc = (p_j >> 14) & 1023
                            rsplat = lax.broadcast(rloc, (16,))
                            for h in range(2):
                                vals = plsc.load_gather(
                                    buf, [c_iota + h * 16, rsplat])
                                vs[pl.ds(e * 32 + h * 16, 16)] = vals
                                ist[pl.ds(e * 32 + h * 16, 16)] = (
                                    lax.broadcast(b_j * 32 + h * 16, (16,))
                                    + c_iota)

                        @pl.when(pos_e >= nw)
                        def _(e=e):
                            for h in range(2):
                                ist[pl.ds(e * 32 + h * 16, 16)] = (
                                    lax.broadcast(
                                        jnp.int32(B * D + e * 32 + h * 16),
                                        (16,)) + c_iota)
                @pl.when(ch < nch - 1)
                def _():
                    scatter_desc(par).start()
                    scatter_desc(par).wait()
                return 0

            lax.fori_loop(0, nch, chunk, 0)
            scatter_desc(par).start()

        issue_window(wid, 0)

        def body(k, _):
            par = k % 2
            w = wid + k * NW
            active = w <= N_FULL

            @pl.when(wid + (k + 1) * NW <= N_FULL)
            def _():
                for p in range(2):
                    @pl.when((k + 1) % 2 == p)
                    def _(p=p):
                        issue_window(wid + (k + 1) * NW, p)

            @pl.when((k >= 2) & (wid + (k - 2) * NW <= N_FULL))
            def _():
                for p in range(2):
                    @pl.when(k % 2 == p)
                    def _(p=p):
                        scatter_desc(p).wait()

            @pl.when(active)
            def _():
                for p in range(2):
                    @pl.when(par == p)
                    def _(p=p):
                        wait_window(w, p)
                        process_window(k, p)

            return 0

        lax.fori_loop(0, N_K + 2, body, 0)

        plsc.subcore_barrier()

        @pl.when(lax.axis_index("s") == 0)
        def _():
            @pl.when(cid == 0)
            def _():
                pltpu.sync_copy(sp_out, out0_hbm)
            @pl.when(cid == 1)
            def _():
                pltpu.sync_copy(sp_out, out1_hbm)

    return lookup


@functools.lru_cache(maxsize=None)
def _make_merge(V, D, B):
    info = plsc.get_sparse_core_info()
    NC, NS = info.num_cores, info.num_subcores
    NW = NC * NS
    EPW = B * D // NW
    BPW = B // NW
    mesh = plsc.VectorSubcoreMesh(core_axis_name="c", subcore_axis_name="s")

    @functools.partial(
        pl.kernel,
        mesh=mesh,
        out_type=jax.ShapeDtypeStruct((B * D,), jnp.float32),
        scratch_types=[
            pltpu.VMEM((BPW,), jnp.int32),
            pltpu.VMEM((EPW,), jnp.float32),
            pltpu.VMEM((EPW,), jnp.float32),
            pltpu.VMEM((EPW,), jnp.float32),
        ],
        compiler_params=pltpu.CompilerParams(
            use_tc_tiling_on_sc=False, needs_layout_passes=False
        ),
    )
    def merge(idx_hbm, h0_hbm, h1_hbm, out_hbm, sp_v, h0_v, h1_v, m_v):
        wid = lax.axis_index("s") * NC + lax.axis_index("c")
        c_iota = lax.iota(jnp.int32, 16)
        base_b = wid * BPW
        base_e = wid * EPW
        pltpu.sync_copy(idx_hbm.at[pl.ds(base_b, BPW)], sp_v)
        pltpu.sync_copy(h0_hbm.at[pl.ds(base_e, EPW)], h0_v)
        pltpu.sync_copy(h1_hbm.at[pl.ds(base_e, EPW)], h1_v)

        def body(g, _):
            lane_e = g * 16 + c_iota
            b_loc = lane_e >> 5
            sp = plsc.load_gather(sp_v, [b_loc])
            owner = (sp >> 9) & (NW - 1)
            v0 = h0_v[pl.ds(g * 16, 16)]
            v1 = h1_v[pl.ds(g * 16, 16)]
            m_v[pl.ds(g * 16, 16)] = jnp.where((owner & 1) == 0, v0, v1)
            return 0

        lax.fori_loop(0, EPW // 16, body, 0)
        pltpu.sync_copy(m_v, out_hbm.at[pl.ds(base_e, EPW)])

    return merge


@jax.jit
def kernel(species, embed_table):
    V, D = embed_table.shape
    (B,) = species.shape
    sp = species.astype(jnp.int32)
    half0, half1 = _make_lookup(V, D, B)(embed_table.T, sp)
    flat = _make_merge(V, D, B)(sp, half0, half1)
    return flat.reshape(B, D)
